# Initial kernel scaffold; baseline (speedup 1.0000x reference)
import jax
import jax.numpy as jnp
from jax.experimental import pallas as pl


def kernel(sentences, embed_weight):
    raise NotImplementedError("write your pallas kernel here")


# speedup vs baseline: 6.1471x; 6.1471x over previous
import functools

import jax
import jax.numpy as jnp
from jax import lax
from jax.experimental import pallas as pl
from jax.experimental.pallas import tpu as pltpu
from jax.experimental.pallas import tpu_sc as plsc

BATCH = 1024
SEQ = 200
EMBED_DIM = 128
B_TOTAL = BATCH * SEQ

NUM_CORES = 2
NUM_SUBCORES = 16
NW = NUM_CORES * NUM_SUBCORES
B_PER_W = B_TOTAL // NW

CHUNK = 128
NCHUNKS = B_PER_W // CHUNK
NBUF = 5
NGROUPS = NCHUNKS // NBUF


@functools.partial(
    pl.kernel,
    mesh=plsc.VectorSubcoreMesh(core_axis_name="c", subcore_axis_name="s"),
    out_type=jax.ShapeDtypeStruct((B_TOTAL, EMBED_DIM), jnp.float32),
    scratch_types=(
        [pltpu.VMEM((NCHUNKS, CHUNK), jnp.int32)]
        + [pltpu.VMEM((CHUNK, EMBED_DIM), jnp.float32) for _ in range(NBUF)]
        + [pltpu.SemaphoreType.DMA for _ in range(2 * NBUF)]
    ),
)
def _embed_gather(table_hbm, idx_hbm, out_hbm, idx_v, *bufs_and_sems):
    bufs = bufs_and_sems[:NBUF]
    gsems = bufs_and_sems[NBUF:2 * NBUF]
    wsems = bufs_and_sems[2 * NBUF:]

    wid = lax.axis_index("s") * NUM_CORES + lax.axis_index("c")
    base = wid * B_PER_W

    pltpu.sync_copy(idx_hbm.at[wid], idx_v)

    def gather(c, b):
        return pltpu.make_async_copy(
            table_hbm.at[idx_v.at[c]], bufs[b], gsems[b])

    def write(c, b):
        return pltpu.make_async_copy(
            bufs[b], out_hbm.at[pl.ds(base + c * CHUNK, CHUNK)], wsems[b])

    for b in range(NBUF):
        gather(b, b).start()

    def group_body(g, carry):
        for b in range(NBUF):
            c = g * NBUF + b
            gather(c, b).wait()
            write(c, b).start()
            write(c, b).wait()
            gather(c + NBUF, b).start()
        return carry

    lax.fori_loop(0, NGROUPS - 1, group_body, 0)

    for b in range(NBUF):
        c = (NGROUPS - 1) * NBUF + b
        gather(c, b).wait()
        write(c, b).start()
        write(c, b).wait()


def kernel(sentences, embed_weight):
    idx = sentences.reshape(NW, NCHUNKS, CHUNK).astype(jnp.int32)
    out = _embed_gather(embed_weight, idx)
    return out.reshape(BATCH, SEQ, EMBED_DIM)
